# baseline (speedup 1.0000x reference)
import functools

import jax
import jax.numpy as jnp
from jax import lax
from jax.experimental import pallas as pl
from jax.experimental.pallas import tpu as pltpu
from jax.experimental.pallas import tpu_sc as plsc

N = 10000
E = 160000
IN_DIM = 128
HID_DIM = 128
LAT_DIM = 64
NUM_GRAPHS = 64
OUT_CLASSES = 10

NC = 2
NS = 16
NW = NC * NS

EPW = E // NW
CH = 40
EPS2 = E // NC // NS
NCH = EPS2 // CH
ZCH = N // CH

@functools.cache
def _sc_mesh():
    return plsc.VectorSubcoreMesh(core_axis_name="c", subcore_axis_name="s",
                                  num_cores=NC, num_subcores=NS)


DCH = 80
DZCH = N // DCH


def _deg_body(dst_hbm, out_hbm, dstbuf, onesbuf, zbuf, acc):
    c = lax.axis_index("c")
    s = lax.axis_index("s")
    z16 = jnp.zeros((16,), jnp.float32)
    ones16 = jnp.ones((16,), jnp.float32)

    for i in range(DCH // 16):
        zbuf[pl.ds(i * 16, 16)] = z16
    for i in range(48 // 16):
        onesbuf[pl.ds(i * 16, 16)] = ones16

    def zacc_step(i, _):
        k = i * NS + s

        @pl.when(k < DZCH)
        def _():
            pltpu.sync_copy(zbuf, acc.at[pl.ds(k * DCH, DCH)])

        return 0

    lax.fori_loop(0, (DZCH + NS - 1) // NS, zacc_step, 0)
    plsc.subcore_barrier()

    w = c * NS + s
    pltpu.sync_copy(dst_hbm.at[w], dstbuf)

    def edge_step(j, _):
        pltpu.sync_copy(onesbuf.at[pl.ds(0, CH)], acc.at[dstbuf.at[j]], add=True)
        return 0

    lax.fori_loop(0, NCH, edge_step, 0)
    plsc.subcore_barrier()

    def drain_step(i, _):
        k = i * NS + s

        @pl.when(k < DZCH)
        def _():
            pltpu.sync_copy(acc.at[pl.ds(k * DCH, DCH)], zbuf)
            pltpu.sync_copy(zbuf, out_hbm.at[pl.ds(c * N + k * DCH, DCH)])

        return 0

    lax.fori_loop(0, (DZCH + NS - 1) // NS, drain_step, 0)


@functools.cache
def _deg_call():
    return pl.kernel(
        _deg_body,
        out_type=jax.ShapeDtypeStruct((NC * N,), jnp.float32),
        mesh=_sc_mesh(),
        scratch_types=[
            pltpu.VMEM((NCH, CH), jnp.int32),
            pltpu.VMEM((48,), jnp.float32),
            pltpu.VMEM((DCH,), jnp.float32),
            pltpu.VMEM_SHARED((N,), jnp.float32),
        ],
    )


NSLOT = 5


def _spmm_body(y_hbm, src_hbm, dst_hbm, out_hbm, srcbuf, dstbuf,
               r0, r1, r2, r3, r4, g0, g1, g2, g3, g4, acc):
    rbufs = (r0, r1, r2, r3, r4)
    gsems = (g0, g1, g2, g3, g4)
    c = lax.axis_index("c")
    s = lax.axis_index("s")
    z16 = jnp.zeros((16,), jnp.float32)

    def zbuf_step(i, _):
        r0[i // 8, pl.ds((i % 8) * 16, 16)] = z16
        return 0

    lax.fori_loop(0, CH * (128 // 16), zbuf_step, 0)

    def zacc_step(i, _):
        k = i * NS + s

        @pl.when(k < ZCH)
        def _():
            pltpu.sync_copy(r0, acc.at[pl.ds(k * CH, CH)])

        return 0

    lax.fori_loop(0, (ZCH + NS - 1) // NS, zacc_step, 0)
    plsc.subcore_barrier()

    w = c * NS + s
    pltpu.sync_copy(src_hbm.at[pl.ds(w * EPS2, EPS2)], srcbuf)
    pltpu.sync_copy(dst_hbm.at[w], dstbuf)

    def gref(j):
        return y_hbm.at[srcbuf.at[pl.ds(j * CH, CH)]]

    for b in range(NSLOT):
        pltpu.async_copy(gref(b), rbufs[b], gsems[b])

    def outer_step(g, _):
        base = g * NSLOT
        for b in range(NSLOT):
            j = base + b
            pltpu.make_async_copy(gref(j), rbufs[b], gsems[b]).wait()
            pltpu.sync_copy(rbufs[b], acc.at[dstbuf.at[j]], add=True)

            @pl.when(j + NSLOT < NCH)
            def _():
                pltpu.async_copy(gref(j + NSLOT), rbufs[b], gsems[b])

        return 0

    lax.fori_loop(0, NCH // NSLOT, outer_step, 0)
    plsc.subcore_barrier()

    def drain_step(i, _):
        k = i * NS + s

        @pl.when(k < ZCH)
        def _():
            pltpu.sync_copy(acc.at[pl.ds(k * CH, CH)], out_hbm.at[c, pl.ds(k * CH, CH)])

        return 0

    lax.fori_loop(0, (ZCH + NS - 1) // NS, drain_step, 0)


@functools.cache
def _spmm_call():
    return pl.kernel(
        _spmm_body,
        out_type=jax.ShapeDtypeStruct((NC, N, HID_DIM), jnp.float32),
        mesh=_sc_mesh(),
        scratch_types=(
            [pltpu.VMEM((EPS2,), jnp.int32), pltpu.VMEM((NCH, CH), jnp.int32)]
            + [pltpu.VMEM((CH, HID_DIM), jnp.float32)] * NSLOT
            + [pltpu.SemaphoreType.DMA] * NSLOT
            + [pltpu.VMEM_SHARED((N, HID_DIM), jnp.float32)]
        ),
    )


BN = 1024
_NBLK = (N + BN - 1) // BN


def _t1_body(deg_ref, x_ref, w1_ref, dinv_ref, y1_ref):
    deg = deg_ref[...]
    ones = jnp.ones((NC, 1), jnp.float32)
    degsum = lax.dot_general(deg, ones, (((0,), (0,)), ((), ())),
                             preferred_element_type=jnp.float32)
    dinv = lax.rsqrt(1.0 + degsum)
    xw = jnp.dot(x_ref[...], w1_ref[...], preferred_element_type=jnp.float32)
    dinv_ref[...] = dinv
    y1_ref[...] = xw * dinv


def _t2_body(dinv_ref, y1_ref, agg_ref, b1_ref, wml_ref, y2_ref):
    dinv = dinv_ref[...]
    a = agg_ref[0] + agg_ref[1] + y1_ref[...]
    h = jnp.maximum(dinv * a + b1_ref[...], 0.0)
    y2_ref[...] = jnp.dot(h, wml_ref[...], preferred_element_type=jnp.float32) * dinv


def _t3_body(dinv_ref, y2_ref, agg_ref, bml_ref, eps_ref, batch_ref,
             wc1_ref, bc1_ref, wc2_ref, bc2_ref,
             mu_ref, lv_ref, zb_ref, logits_ref, sums_ref, cnts_ref):
    i = pl.program_id(0)
    dinv = dinv_ref[...]
    mulv = dinv * (agg_ref[0] + agg_ref[1] + y2_ref[...]) + bml_ref[...]
    mu = mulv[:, :LAT_DIM]
    lv = mulv[:, LAT_DIM:]
    mu_ref[...] = mu
    lv_ref[...] = lv
    z = mu + eps_ref[...] * jnp.exp(0.5 * lv)
    zb_ref[...] = z.astype(jnp.bfloat16)

    @pl.when(i == 0)
    def _():
        sums_ref[...] = jnp.zeros_like(sums_ref)
        cnts_ref[...] = jnp.zeros_like(cnts_ref)

    b = batch_ref[...]
    gids = lax.broadcasted_iota(jnp.int32, (BN, NUM_GRAPHS), 1)
    rows = lax.broadcasted_iota(jnp.int32, (BN, NUM_GRAPHS), 0) + i * BN
    onehot = ((b == gids) & (rows < N)).astype(jnp.float32)
    rowmask = (lax.broadcasted_iota(jnp.int32, (BN, 1), 0) + i * BN) < N
    zm = jnp.where(rowmask, z, 0.0)
    sums_ref[...] += lax.dot_general(onehot, zm, (((0,), (0,)), ((), ())),
                                     preferred_element_type=jnp.float32)
    cnts_ref[...] += lax.dot_general(onehot, jnp.ones((BN, 1), jnp.float32),
                                     (((0,), (0,)), ((), ())),
                                     preferred_element_type=jnp.float32)
    ge = sums_ref[...] / jnp.maximum(cnts_ref[...], 1.0)
    hc = jnp.maximum(jnp.dot(ge, wc1_ref[...], preferred_element_type=jnp.float32)
                     + bc1_ref[...], 0.0)
    logits_ref[...] = jnp.dot(hc, wc2_ref[...], preferred_element_type=jnp.float32) + bc2_ref[...]


BI = 1024
BJ = 4096


def _t4_body(zi_ref, zj_ref, out_ref):
    t = lax.dot_general(zi_ref[...], zj_ref[...], (((1,), (1,)), ((), ())),
                        preferred_element_type=jnp.float32)
    out_ref[...] = 0.5 + 0.5 * jnp.tanh(0.5 * t)


def kernel(x, edge_index, batch, eps, W1, b1, Wmu, bmu, Wlv, blv, Wc1, bc1, Wc2, bc2):
    f32 = jnp.float32
    src = edge_index[0].astype(jnp.int32)
    dst = edge_index[1].astype(jnp.int32)
    dst40 = dst.reshape(NW, NCH, CH)

    Wml = jnp.concatenate([Wmu, Wlv], axis=1)
    bml = jnp.concatenate([bmu, blv]).reshape(1, HID_DIM)
    b1r = b1.reshape(1, HID_DIM)
    bc1r = bc1.reshape(1, 64)
    bc2r = bc2.reshape(1, OUT_CLASSES)
    batch2d = batch.astype(jnp.int32).reshape(N, 1)

    deg_part = _deg_call()(dst40).reshape(NC, N)

    dinv, Y1 = pl.pallas_call(
        _t1_body,
        grid=(_NBLK,),
        in_specs=[
            pl.BlockSpec((NC, BN), lambda i: (0, i)),
            pl.BlockSpec((BN, IN_DIM), lambda i: (i, 0)),
            pl.BlockSpec((IN_DIM, HID_DIM), lambda i: (0, 0)),
        ],
        out_specs=[
            pl.BlockSpec((BN, 1), lambda i: (i, 0)),
            pl.BlockSpec((BN, HID_DIM), lambda i: (i, 0)),
        ],
        out_shape=[
            jax.ShapeDtypeStruct((N, 1), f32),
            jax.ShapeDtypeStruct((N, HID_DIM), f32),
        ],
    )(deg_part, x, W1)

    agg1 = _spmm_call()(Y1, src, dst40)

    Y2 = pl.pallas_call(
        _t2_body,
        grid=(_NBLK,),
        in_specs=[
            pl.BlockSpec((BN, 1), lambda i: (i, 0)),
            pl.BlockSpec((BN, HID_DIM), lambda i: (i, 0)),
            pl.BlockSpec((NC, BN, HID_DIM), lambda i: (0, i, 0)),
            pl.BlockSpec((1, HID_DIM), lambda i: (0, 0)),
            pl.BlockSpec((HID_DIM, HID_DIM), lambda i: (0, 0)),
        ],
        out_specs=pl.BlockSpec((BN, HID_DIM), lambda i: (i, 0)),
        out_shape=jax.ShapeDtypeStruct((N, HID_DIM), f32),
    )(dinv, Y1, agg1, b1r, Wml)

    agg2 = _spmm_call()(Y2, src, dst40)

    mu, logvar, zb, class_logits = pl.pallas_call(
        _t3_body,
        grid=(_NBLK,),
        in_specs=[
            pl.BlockSpec((BN, 1), lambda i: (i, 0)),
            pl.BlockSpec((BN, HID_DIM), lambda i: (i, 0)),
            pl.BlockSpec((NC, BN, HID_DIM), lambda i: (0, i, 0)),
            pl.BlockSpec((1, HID_DIM), lambda i: (0, 0)),
            pl.BlockSpec((BN, LAT_DIM), lambda i: (i, 0)),
            pl.BlockSpec((BN, 1), lambda i: (i, 0)),
            pl.BlockSpec((LAT_DIM, 64), lambda i: (0, 0)),
            pl.BlockSpec((1, 64), lambda i: (0, 0)),
            pl.BlockSpec((64, OUT_CLASSES), lambda i: (0, 0)),
            pl.BlockSpec((1, OUT_CLASSES), lambda i: (0, 0)),
        ],
        out_specs=[
            pl.BlockSpec((BN, LAT_DIM), lambda i: (i, 0)),
            pl.BlockSpec((BN, LAT_DIM), lambda i: (i, 0)),
            pl.BlockSpec((BN, LAT_DIM), lambda i: (i, 0)),
            pl.BlockSpec((NUM_GRAPHS, OUT_CLASSES), lambda i: (0, 0)),
        ],
        out_shape=[
            jax.ShapeDtypeStruct((N, LAT_DIM), f32),
            jax.ShapeDtypeStruct((N, LAT_DIM), f32),
            jax.ShapeDtypeStruct((N, LAT_DIM), jnp.bfloat16),
            jax.ShapeDtypeStruct((NUM_GRAPHS, OUT_CLASSES), f32),
        ],
        scratch_shapes=[
            pltpu.VMEM((NUM_GRAPHS, LAT_DIM), f32),
            pltpu.VMEM((NUM_GRAPHS, 1), f32),
        ],
    )(dinv, Y2, agg2, bml, eps, batch2d, Wc1, bc1r, Wc2, bc2r)

    adj_pred = pl.pallas_call(
        _t4_body,
        grid=((N + BI - 1) // BI, (N + BJ - 1) // BJ),
        in_specs=[
            pl.BlockSpec((BI, LAT_DIM), lambda i, j: (i, 0)),
            pl.BlockSpec((BJ, LAT_DIM), lambda i, j: (j, 0)),
        ],
        out_specs=pl.BlockSpec((BI, BJ), lambda i, j: (i, j)),
        out_shape=jax.ShapeDtypeStruct((N, N), f32),
    )(zb, zb)

    return (adj_pred, mu, logvar, class_logits)


# speedup vs baseline: 1.0478x; 1.0478x over previous
import functools

import jax
import jax.numpy as jnp
from jax import lax
from jax.experimental import pallas as pl
from jax.experimental.pallas import tpu as pltpu
from jax.experimental.pallas import tpu_sc as plsc

N = 10000
E = 160000
IN_DIM = 128
HID_DIM = 128
LAT_DIM = 64
NUM_GRAPHS = 64
OUT_CLASSES = 10

NC = 2
NS = 16
NW = NC * NS

EPW = E // NW
CH = 40
EPS2 = E // NC // NS
NCH = EPS2 // CH
ZCH = N // CH

@functools.cache
def _sc_mesh():
    return plsc.VectorSubcoreMesh(core_axis_name="c", subcore_axis_name="s",
                                  num_cores=NC, num_subcores=NS)


DCH = 80
DZCH = N // DCH


def _deg_body(dst_hbm, out_hbm, dstbuf, onesbuf, zbuf, acc):
    c = lax.axis_index("c")
    s = lax.axis_index("s")
    z16 = jnp.zeros((16,), jnp.float32)
    ones16 = jnp.ones((16,), jnp.float32)

    for i in range(DCH // 16):
        zbuf[pl.ds(i * 16, 16)] = z16
    for i in range(48 // 16):
        onesbuf[pl.ds(i * 16, 16)] = ones16

    def zacc_step(i, _):
        k = i * NS + s

        @pl.when(k < DZCH)
        def _():
            pltpu.sync_copy(zbuf, acc.at[pl.ds(k * DCH, DCH)])

        return 0

    lax.fori_loop(0, (DZCH + NS - 1) // NS, zacc_step, 0)
    plsc.subcore_barrier()

    w = c * NS + s
    pltpu.sync_copy(dst_hbm.at[w], dstbuf)

    def edge_step(j, _):
        pltpu.sync_copy(onesbuf.at[pl.ds(0, CH)], acc.at[dstbuf.at[j]], add=True)
        return 0

    lax.fori_loop(0, NCH, edge_step, 0)
    plsc.subcore_barrier()

    def drain_step(i, _):
        k = i * NS + s

        @pl.when(k < DZCH)
        def _():
            pltpu.sync_copy(acc.at[pl.ds(k * DCH, DCH)], zbuf)
            pltpu.sync_copy(zbuf, out_hbm.at[pl.ds(c * N + k * DCH, DCH)])

        return 0

    lax.fori_loop(0, (DZCH + NS - 1) // NS, drain_step, 0)


@functools.cache
def _deg_call():
    return pl.kernel(
        _deg_body,
        out_type=jax.ShapeDtypeStruct((NC * N,), jnp.float32),
        mesh=_sc_mesh(),
        scratch_types=[
            pltpu.VMEM((NCH, CH), jnp.int32),
            pltpu.VMEM((48,), jnp.float32),
            pltpu.VMEM((DCH,), jnp.float32),
            pltpu.VMEM_SHARED((N,), jnp.float32),
        ],
    )


NSLOT = 5


def _spmm_body(y_hbm, src_hbm, dst_hbm, out_hbm, srcbuf, dstbuf,
               r0, r1, r2, r3, r4, g0, g1, g2, g3, g4, acc):
    rbufs = (r0, r1, r2, r3, r4)
    gsems = (g0, g1, g2, g3, g4)
    c = lax.axis_index("c")
    s = lax.axis_index("s")
    z16 = jnp.zeros((16,), jnp.float32)

    def zbuf_step(i, _):
        r0[i // 8, pl.ds((i % 8) * 16, 16)] = z16
        return 0

    lax.fori_loop(0, CH * (128 // 16), zbuf_step, 0)

    def zacc_step(i, _):
        k = i * NS + s

        @pl.when(k < ZCH)
        def _():
            pltpu.sync_copy(r0, acc.at[pl.ds(k * CH, CH)])

        return 0

    lax.fori_loop(0, (ZCH + NS - 1) // NS, zacc_step, 0)
    plsc.subcore_barrier()

    w = c * NS + s
    pltpu.sync_copy(src_hbm.at[pl.ds(w * EPS2, EPS2)], srcbuf)
    pltpu.sync_copy(dst_hbm.at[w], dstbuf)

    def gref(j):
        return y_hbm.at[srcbuf.at[pl.ds(j * CH, CH)]]

    for b in range(NSLOT):
        pltpu.async_copy(gref(b), rbufs[b], gsems[b])

    def outer_step(g, _):
        base = g * NSLOT
        for b in range(NSLOT):
            j = base + b
            pltpu.make_async_copy(gref(j), rbufs[b], gsems[b]).wait()
            pltpu.sync_copy(rbufs[b], acc.at[dstbuf.at[j]], add=True)

            @pl.when(j + NSLOT < NCH)
            def _():
                pltpu.async_copy(gref(j + NSLOT), rbufs[b], gsems[b])

        return 0

    lax.fori_loop(0, NCH // NSLOT, outer_step, 0)
    plsc.subcore_barrier()

    def drain_step(i, _):
        k = i * NS + s

        @pl.when(k < ZCH)
        def _():
            pltpu.sync_copy(acc.at[pl.ds(k * CH, CH)], out_hbm.at[c, pl.ds(k * CH, CH)])

        return 0

    lax.fori_loop(0, (ZCH + NS - 1) // NS, drain_step, 0)


@functools.cache
def _spmm_call():
    return pl.kernel(
        _spmm_body,
        out_type=jax.ShapeDtypeStruct((NC, N, HID_DIM), jnp.float32),
        mesh=_sc_mesh(),
        scratch_types=(
            [pltpu.VMEM((EPS2,), jnp.int32), pltpu.VMEM((NCH, CH), jnp.int32)]
            + [pltpu.VMEM((CH, HID_DIM), jnp.float32)] * NSLOT
            + [pltpu.SemaphoreType.DMA] * NSLOT
            + [pltpu.VMEM_SHARED((N, HID_DIM), jnp.float32)]
        ),
    )


BN = 1024
_NBLK = (N + BN - 1) // BN


def _t1_body(deg_ref, x_ref, w1_ref, dinv_ref, y1_ref):
    deg = deg_ref[...]
    ones = jnp.ones((NC, 1), jnp.float32)
    degsum = lax.dot_general(deg, ones, (((0,), (0,)), ((), ())),
                             preferred_element_type=jnp.float32)
    dinv = lax.rsqrt(1.0 + degsum)
    xw = jnp.dot(x_ref[...], w1_ref[...], preferred_element_type=jnp.float32)
    dinv_ref[...] = dinv
    y1_ref[...] = xw * dinv


def _t2_body(dinv_ref, y1_ref, agg_ref, b1_ref, wml_ref, y2_ref):
    dinv = dinv_ref[...]
    a = agg_ref[0] + agg_ref[1] + y1_ref[...]
    h = jnp.maximum(dinv * a + b1_ref[...], 0.0)
    y2_ref[...] = jnp.dot(h, wml_ref[...], preferred_element_type=jnp.float32) * dinv


def _t3_body(dinv_ref, y2_ref, agg_ref, bml_ref, eps_ref, batch_ref,
             wc1_ref, bc1_ref, wc2_ref, bc2_ref,
             mu_ref, lv_ref, zb_ref, logits_ref, sums_ref, cnts_ref):
    i = pl.program_id(0)
    dinv = dinv_ref[...]
    mulv = dinv * (agg_ref[0] + agg_ref[1] + y2_ref[...]) + bml_ref[...]
    mu = mulv[:, :LAT_DIM]
    lv = mulv[:, LAT_DIM:]
    mu_ref[...] = mu
    lv_ref[...] = lv
    z = mu + eps_ref[...] * jnp.exp(0.5 * lv)
    zb_ref[...] = z.astype(jnp.bfloat16)

    @pl.when(i == 0)
    def _():
        sums_ref[...] = jnp.zeros_like(sums_ref)
        cnts_ref[...] = jnp.zeros_like(cnts_ref)

    b = batch_ref[...]
    gids = lax.broadcasted_iota(jnp.int32, (BN, NUM_GRAPHS), 1)
    rows = lax.broadcasted_iota(jnp.int32, (BN, NUM_GRAPHS), 0) + i * BN
    onehot = ((b == gids) & (rows < N)).astype(jnp.float32)
    rowmask = (lax.broadcasted_iota(jnp.int32, (BN, 1), 0) + i * BN) < N
    zm = jnp.where(rowmask, z, 0.0)
    sums_ref[...] += lax.dot_general(onehot, zm, (((0,), (0,)), ((), ())),
                                     preferred_element_type=jnp.float32)
    cnts_ref[...] += lax.dot_general(onehot, jnp.ones((BN, 1), jnp.float32),
                                     (((0,), (0,)), ((), ())),
                                     preferred_element_type=jnp.float32)
    ge = sums_ref[...] / jnp.maximum(cnts_ref[...], 1.0)
    hc = jnp.maximum(jnp.dot(ge, wc1_ref[...], preferred_element_type=jnp.float32)
                     + bc1_ref[...], 0.0)
    logits_ref[...] = jnp.dot(hc, wc2_ref[...], preferred_element_type=jnp.float32) + bc2_ref[...]


BI = 2048
BJ = 2048


def _t4_body(zi_ref, zj_ref, out_ref):
    t = lax.dot_general(zi_ref[...], zj_ref[...], (((1,), (1,)), ((), ())),
                        preferred_element_type=jnp.float32)
    out_ref[...] = 0.5 + 0.5 * jnp.tanh(0.5 * t)


def kernel(x, edge_index, batch, eps, W1, b1, Wmu, bmu, Wlv, blv, Wc1, bc1, Wc2, bc2):
    f32 = jnp.float32
    src = edge_index[0].astype(jnp.int32)
    dst = edge_index[1].astype(jnp.int32)
    dst40 = dst.reshape(NW, NCH, CH)

    Wml = jnp.concatenate([Wmu, Wlv], axis=1)
    bml = jnp.concatenate([bmu, blv]).reshape(1, HID_DIM)
    b1r = b1.reshape(1, HID_DIM)
    bc1r = bc1.reshape(1, 64)
    bc2r = bc2.reshape(1, OUT_CLASSES)
    batch2d = batch.astype(jnp.int32).reshape(N, 1)

    deg_part = _deg_call()(dst40).reshape(NC, N)

    dinv, Y1 = pl.pallas_call(
        _t1_body,
        grid=(_NBLK,),
        in_specs=[
            pl.BlockSpec((NC, BN), lambda i: (0, i)),
            pl.BlockSpec((BN, IN_DIM), lambda i: (i, 0)),
            pl.BlockSpec((IN_DIM, HID_DIM), lambda i: (0, 0)),
        ],
        out_specs=[
            pl.BlockSpec((BN, 1), lambda i: (i, 0)),
            pl.BlockSpec((BN, HID_DIM), lambda i: (i, 0)),
        ],
        out_shape=[
            jax.ShapeDtypeStruct((N, 1), f32),
            jax.ShapeDtypeStruct((N, HID_DIM), f32),
        ],
    )(deg_part, x, W1)

    agg1 = _spmm_call()(Y1, src, dst40)

    Y2 = pl.pallas_call(
        _t2_body,
        grid=(_NBLK,),
        in_specs=[
            pl.BlockSpec((BN, 1), lambda i: (i, 0)),
            pl.BlockSpec((BN, HID_DIM), lambda i: (i, 0)),
            pl.BlockSpec((NC, BN, HID_DIM), lambda i: (0, i, 0)),
            pl.BlockSpec((1, HID_DIM), lambda i: (0, 0)),
            pl.BlockSpec((HID_DIM, HID_DIM), lambda i: (0, 0)),
        ],
        out_specs=pl.BlockSpec((BN, HID_DIM), lambda i: (i, 0)),
        out_shape=jax.ShapeDtypeStruct((N, HID_DIM), f32),
    )(dinv, Y1, agg1, b1r, Wml)

    agg2 = _spmm_call()(Y2, src, dst40)

    mu, logvar, zb, class_logits = pl.pallas_call(
        _t3_body,
        grid=(_NBLK,),
        in_specs=[
            pl.BlockSpec((BN, 1), lambda i: (i, 0)),
            pl.BlockSpec((BN, HID_DIM), lambda i: (i, 0)),
            pl.BlockSpec((NC, BN, HID_DIM), lambda i: (0, i, 0)),
            pl.BlockSpec((1, HID_DIM), lambda i: (0, 0)),
            pl.BlockSpec((BN, LAT_DIM), lambda i: (i, 0)),
            pl.BlockSpec((BN, 1), lambda i: (i, 0)),
            pl.BlockSpec((LAT_DIM, 64), lambda i: (0, 0)),
            pl.BlockSpec((1, 64), lambda i: (0, 0)),
            pl.BlockSpec((64, OUT_CLASSES), lambda i: (0, 0)),
            pl.BlockSpec((1, OUT_CLASSES), lambda i: (0, 0)),
        ],
        out_specs=[
            pl.BlockSpec((BN, LAT_DIM), lambda i: (i, 0)),
            pl.BlockSpec((BN, LAT_DIM), lambda i: (i, 0)),
            pl.BlockSpec((BN, LAT_DIM), lambda i: (i, 0)),
            pl.BlockSpec((NUM_GRAPHS, OUT_CLASSES), lambda i: (0, 0)),
        ],
        out_shape=[
            jax.ShapeDtypeStruct((N, LAT_DIM), f32),
            jax.ShapeDtypeStruct((N, LAT_DIM), f32),
            jax.ShapeDtypeStruct((N, LAT_DIM), jnp.bfloat16),
            jax.ShapeDtypeStruct((NUM_GRAPHS, OUT_CLASSES), f32),
        ],
        scratch_shapes=[
            pltpu.VMEM((NUM_GRAPHS, LAT_DIM), f32),
            pltpu.VMEM((NUM_GRAPHS, 1), f32),
        ],
    )(dinv, Y2, agg2, bml, eps, batch2d, Wc1, bc1r, Wc2, bc2r)

    adj_pred = pl.pallas_call(
        _t4_body,
        grid=((N + BI - 1) // BI, (N + BJ - 1) // BJ),
        in_specs=[
            pl.BlockSpec((BI, LAT_DIM), lambda i, j: (i, 0)),
            pl.BlockSpec((BJ, LAT_DIM), lambda i, j: (j, 0)),
        ],
        out_specs=pl.BlockSpec((BI, BJ), lambda i, j: (i, j)),
        out_shape=jax.ShapeDtypeStruct((N, N), f32),
    )(zb, zb)

    return (adj_pred, mu, logvar, class_logits)
